# baseline (speedup 1.0000x reference)
import functools

import jax
import jax.numpy as jnp
from jax import lax
from jax.experimental import pallas as pl
from jax.experimental.pallas import tpu as pltpu
from jax.experimental.pallas import tpu_sc as plsc

_N = 10000
_E = 160000
_NC, _NS = 2, 16
_NW = _NC * _NS
_BK = 128
_EPAD = 163840
_FAST = 1
_EPW_F = 8192
_EPW_S = 2048
_EPW_D = 5120
_NACC = 10240
_ZR = _NACC // _NS

_mesh = plsc.VectorSubcoreMesh(core_axis_name="c", subcore_axis_name="s")


def _deg_body(dst_hbm, out_hbm, dst_v, ones_v, zb, acc, sem):
    del sem
    c = lax.axis_index("c")
    s = lax.axis_index("s")
    base = (s * _NC + c) * _EPW_D
    nb = _EPW_D // _BK
    r0 = s * _ZR

    def fill(i, carry):
        for j in range(8):
            zb[i, pl.ds(j * 16, 16)] = jnp.zeros((16,), jnp.float32)
            ones_v[i, pl.ds(j * 16, 16)] = jnp.ones((16,), jnp.float32)
        return carry
    lax.fori_loop(0, _BK, fill, None)

    for k in range(_ZR // _BK):
        pltpu.sync_copy(zb, acc.at[pl.ds(r0 + k * _BK, _BK)])
    plsc.subcore_barrier()

    def estep(i, carry):
        pltpu.sync_copy(dst_hbm.at[pl.ds(base + i * _BK, _BK)], dst_v)
        pltpu.sync_copy(ones_v, acc.at[dst_v], add=True)
        return carry
    lax.fori_loop(0, nb, estep, None)

    plsc.subcore_barrier()

    @pl.when(c == 0)
    def _():
        pltpu.sync_copy(acc.at[pl.ds(r0, _ZR)], out_hbm.at[0, pl.ds(r0, _ZR)])

    @pl.when(c == 1)
    def _():
        pltpu.sync_copy(acc.at[pl.ds(r0, _ZR)], out_hbm.at[1, pl.ds(r0, _ZR)])


def _make_deg():
    return pl.kernel(
        _deg_body,
        out_type=jax.ShapeDtypeStruct((_NC, _NACC, 128), jnp.float32),
        mesh=_mesh,
        scratch_types=[
            pltpu.VMEM((_BK,), jnp.int32),
            pltpu.VMEM((_BK, 128), jnp.float32),
            pltpu.VMEM((_BK, 128), jnp.float32),
            pltpu.VMEM_SHARED((_NACC, 128), jnp.float32),
            pltpu.SemaphoreType.DMA,
        ],
    )


def _agg_body(nchunks, fc, *refs):
    g_refs = refs[:nchunks]
    src_hbm, dst_hbm, out_hbm = refs[nchunks:nchunks + 3]
    src0, dst0, src1, dst1, rows0, rows1, acc, sem0, sem1 = refs[nchunks + 3:]
    c = lax.axis_index("c")
    s = lax.axis_index("s")
    is_fast = c == _FAST
    base = jnp.where(is_fast, s * _EPW_F, _NS * _EPW_F + s * _EPW_S)
    nb = jnp.where(is_fast, _EPW_F // _BK, _EPW_S // _BK)
    r0 = s * _ZR

    for ci in range(nchunks):
        g_hbm = g_refs[ci]

        def zfill(i, carry):
            for j in range(fc // 16):
                rows0[i, pl.ds(j * 16, 16)] = jnp.zeros((16,), jnp.float32)
            return carry
        lax.fori_loop(0, _BK, zfill, None)
        for k in range(_ZR // _BK):
            pltpu.sync_copy(rows0, acc.at[pl.ds(r0 + k * _BK, _BK)])
        plsc.subcore_barrier()

        pltpu.sync_copy(src_hbm.at[pl.ds(base, _BK)], src0)
        pltpu.sync_copy(dst_hbm.at[pl.ds(base, _BK)], dst0)
        pltpu.async_copy(g_hbm.at[src0], rows0, sem0)

        @pl.loop(0, nb, step=2)
        def _(i):
            pltpu.sync_copy(src_hbm.at[pl.ds(base + (i + 1) * _BK, _BK)],
                            src1)
            pltpu.sync_copy(dst_hbm.at[pl.ds(base + (i + 1) * _BK, _BK)],
                            dst1)
            pltpu.async_copy(g_hbm.at[src1], rows1, sem1)
            pltpu.make_async_copy(g_hbm.at[src0], rows0, sem0).wait()
            pltpu.sync_copy(rows0, acc.at[dst0], add=True)

            @pl.when(i + 2 < nb)
            def _():
                pltpu.sync_copy(src_hbm.at[pl.ds(base + (i + 2) * _BK, _BK)],
                                src0)
                pltpu.sync_copy(dst_hbm.at[pl.ds(base + (i + 2) * _BK, _BK)],
                                dst0)
                pltpu.async_copy(g_hbm.at[src0], rows0, sem0)

            pltpu.make_async_copy(g_hbm.at[src1], rows1, sem1).wait()
            pltpu.sync_copy(rows1, acc.at[dst1], add=True)

        plsc.subcore_barrier()

        @pl.when(c == 0)
        def _(ci=ci):
            pltpu.sync_copy(acc.at[pl.ds(r0, _ZR)],
                            out_hbm.at[ci, 0, pl.ds(r0, _ZR)])

        @pl.when(c == 1)
        def _(ci=ci):
            pltpu.sync_copy(acc.at[pl.ds(r0, _ZR)],
                            out_hbm.at[ci, 1, pl.ds(r0, _ZR)])


def _make_agg(nchunks, fc):
    return pl.kernel(
        functools.partial(_agg_body, nchunks, fc),
        out_type=jax.ShapeDtypeStruct((nchunks, _NC, _NACC, fc), jnp.float32),
        mesh=_mesh,
        scratch_types=[
            pltpu.VMEM((_BK,), jnp.int32),
            pltpu.VMEM((_BK,), jnp.int32),
            pltpu.VMEM((_BK,), jnp.int32),
            pltpu.VMEM((_BK,), jnp.int32),
            pltpu.VMEM((_BK, fc), jnp.float32),
            pltpu.VMEM((_BK, fc), jnp.float32),
            pltpu.VMEM_SHARED((_NACC, fc), jnp.float32),
            pltpu.SemaphoreType.DMA,
            pltpu.SemaphoreType.DMA,
        ],
    )


def _disg0_kernel(degp, x, dis, g0):
    d = degp[0, :, 0:1] + degp[1, :, 0:1] + 1.0
    r = lax.rsqrt(d)
    dis[...] = r
    g0[...] = r * x[...]


def _mm1_kernel(p, g0, dis, w, b, o):
    a = dis[...] * (p[0, 0] + p[0, 1] + g0[...])
    h = jnp.dot(a.astype(jnp.bfloat16), w[...],
                preferred_element_type=jnp.float32) + b[...]
    o[...] = jnp.maximum(h, 0.0).astype(jnp.bfloat16)


def _mm2_kernel(h1, w, dis, o):
    o[0] = dis[...] * jnp.dot(h1[...], w[...],
                              preferred_element_type=jnp.float32)


def _fin2_kernel(p, g2, dis, b, o):
    t = p[0, 0] + p[0, 1] + g2[0]
    o[...] = jnp.maximum(dis[...] * t + b[0], 0.0).astype(jnp.bfloat16)


def _mm3_kernel(h2, w, dis, o):
    o[...] = dis[...] * jnp.dot(h2[...], w[...],
                                preferred_element_type=jnp.float32)


def _mm4_kernel(p, g3, dis, b, w, o):
    h3 = jnp.maximum(dis[...] * (p[0, 0] + p[0, 1] + g3[...]) + b[...], 0.0)
    o[...] = dis[...] * jnp.dot(h3.astype(jnp.bfloat16), w[...],
                                preferred_element_type=jnp.float32)


def _final_kernel(p, g4, dis, b, o):
    t = dis[...] * (p[0, 0] + p[0, 1] + g4[...])
    o[...] = t[:, :64] + b[...]


def kernel(x, edge_index, W1, b1, W2, b2, W3, b3, W4, b4):
    f32 = jnp.float32
    src = edge_index[0]
    dst = edge_index[1]

    pad = _EPAD - _E
    src_p = jnp.concatenate([src, jnp.zeros((pad,), jnp.int32)])
    dst_p = jnp.concatenate([dst, jnp.full((pad,), _N, jnp.int32)])

    bf16 = jnp.bfloat16
    W1b = W1.astype(bf16)
    W2p = jnp.pad(W2, ((0, 0), (0, 12))).astype(bf16)
    b2p = jnp.pad(b2, (0, 12)).reshape(4, 1, 128)
    W3p = jnp.pad(W3, ((0, 12), (0, 28))).astype(bf16)
    b3p = jnp.pad(b3, (0, 28)).reshape(1, 128)
    W4p = jnp.pad(W4, ((0, 28), (0, 64))).astype(bf16)
    b1r = b1.reshape(1, 2000)
    b4r = b4.reshape(1, 64)

    degp = _make_deg()(dst_p)
    dis, g0 = pl.pallas_call(
        _disg0_kernel,
        grid=(25,),
        in_specs=[
            pl.BlockSpec((2, 400, 128), lambda m: (0, m, 0)),
            pl.BlockSpec((400, 128), lambda m: (m, 0)),
        ],
        out_specs=[
            pl.BlockSpec((400, 1), lambda m: (m, 0)),
            pl.BlockSpec((400, 128), lambda m: (m, 0)),
        ],
        out_shape=[
            jax.ShapeDtypeStruct((_N, 1), f32),
            jax.ShapeDtypeStruct((_N, 128), f32),
        ],
    )(degp, x)

    p1 = _make_agg(1, 128)(g0, src_p, dst_p)
    h1 = pl.pallas_call(
        _mm1_kernel,
        grid=(25,),
        in_specs=[
            pl.BlockSpec((1, 2, 400, 128), lambda m: (0, 0, m, 0)),
            pl.BlockSpec((400, 128), lambda m: (m, 0)),
            pl.BlockSpec((400, 1), lambda m: (m, 0)),
            pl.BlockSpec((128, 2000), lambda m: (0, 0)),
            pl.BlockSpec((1, 2000), lambda m: (0, 0)),
        ],
        out_specs=pl.BlockSpec((400, 2000), lambda m: (m, 0)),
        out_shape=jax.ShapeDtypeStruct((_N, 2000), bf16),
    )(p1, g0, dis, W1b, b1r)

    g2 = pl.pallas_call(
        _mm2_kernel,
        grid=(25, 4),
        in_specs=[
            pl.BlockSpec((400, 2000), lambda m, c: (m, 0)),
            pl.BlockSpec((2000, 128), lambda m, c: (0, c)),
            pl.BlockSpec((400, 1), lambda m, c: (m, 0)),
        ],
        out_specs=pl.BlockSpec((1, 400, 128), lambda m, c: (c, m, 0)),
        out_shape=jax.ShapeDtypeStruct((4, _N, 128), f32),
    )(h1, W2p, dis)
    p2 = _make_agg(4, 128)(g2[0], g2[1], g2[2], g2[3], src_p, dst_p)
    h2 = pl.pallas_call(
        _fin2_kernel,
        grid=(4, 25),
        in_specs=[
            pl.BlockSpec((1, 2, 400, 128), lambda c, m: (c, 0, m, 0)),
            pl.BlockSpec((1, 400, 128), lambda c, m: (c, m, 0)),
            pl.BlockSpec((400, 1), lambda c, m: (m, 0)),
            pl.BlockSpec((1, 1, 128), lambda c, m: (c, 0, 0)),
        ],
        out_specs=pl.BlockSpec((400, 128), lambda c, m: (m, c)),
        out_shape=jax.ShapeDtypeStruct((_N, 512), bf16),
    )(p2, g2, dis, b2p)

    g3 = pl.pallas_call(
        _mm3_kernel,
        grid=(25,),
        in_specs=[
            pl.BlockSpec((400, 512), lambda m: (m, 0)),
            pl.BlockSpec((512, 128), lambda m: (0, 0)),
            pl.BlockSpec((400, 1), lambda m: (m, 0)),
        ],
        out_specs=pl.BlockSpec((400, 128), lambda m: (m, 0)),
        out_shape=jax.ShapeDtypeStruct((_N, 128), f32),
    )(h2, W3p, dis)
    p3 = _make_agg(1, 128)(g3, src_p, dst_p)

    g4 = pl.pallas_call(
        _mm4_kernel,
        grid=(25,),
        in_specs=[
            pl.BlockSpec((1, 2, 400, 128), lambda m: (0, 0, m, 0)),
            pl.BlockSpec((400, 128), lambda m: (m, 0)),
            pl.BlockSpec((400, 1), lambda m: (m, 0)),
            pl.BlockSpec((1, 128), lambda m: (0, 0)),
            pl.BlockSpec((128, 128), lambda m: (0, 0)),
        ],
        out_specs=pl.BlockSpec((400, 128), lambda m: (m, 0)),
        out_shape=jax.ShapeDtypeStruct((_N, 128), f32),
    )(p3, g3, dis, b3p, W4p)
    p4 = _make_agg(1, 128)(g4, src_p, dst_p)

    out = pl.pallas_call(
        _final_kernel,
        grid=(25,),
        in_specs=[
            pl.BlockSpec((1, 2, 400, 128), lambda m: (0, 0, m, 0)),
            pl.BlockSpec((400, 128), lambda m: (m, 0)),
            pl.BlockSpec((400, 1), lambda m: (m, 0)),
            pl.BlockSpec((1, 64), lambda m: (0, 0)),
        ],
        out_specs=pl.BlockSpec((400, 64), lambda m: (m, 0)),
        out_shape=jax.ShapeDtypeStruct((_N, 64), f32),
    )(p4, g4, dis, b4r)
    return out


# speedup vs baseline: 1.0297x; 1.0297x over previous
import functools

import jax
import jax.numpy as jnp
from jax import lax
from jax.experimental import pallas as pl
from jax.experimental.pallas import tpu as pltpu
from jax.experimental.pallas import tpu_sc as plsc

_N = 10000
_E = 160000
_NC, _NS = 2, 16
_NW = _NC * _NS
_BK = 128
_EPAD = 163840
_FAST = 1
_EPW_F = 8960
_EPW_S = 1280
_EPW_D = 5120
_NACC = 10240
_ZR = _NACC // _NS

_mesh = plsc.VectorSubcoreMesh(core_axis_name="c", subcore_axis_name="s")


def _deg_body(dst_hbm, out_hbm, dst_v, ones_v, zb, acc, sem):
    del sem
    c = lax.axis_index("c")
    s = lax.axis_index("s")
    base = (s * _NC + c) * _EPW_D
    nb = _EPW_D // _BK
    r0 = s * _ZR

    def fill(i, carry):
        for j in range(8):
            zb[i, pl.ds(j * 16, 16)] = jnp.zeros((16,), jnp.float32)
            ones_v[i, pl.ds(j * 16, 16)] = jnp.ones((16,), jnp.float32)
        return carry
    lax.fori_loop(0, _BK, fill, None)

    for k in range(_ZR // _BK):
        pltpu.sync_copy(zb, acc.at[pl.ds(r0 + k * _BK, _BK)])
    plsc.subcore_barrier()

    def estep(i, carry):
        pltpu.sync_copy(dst_hbm.at[pl.ds(base + i * _BK, _BK)], dst_v)
        pltpu.sync_copy(ones_v, acc.at[dst_v], add=True)
        return carry
    lax.fori_loop(0, nb, estep, None)

    plsc.subcore_barrier()

    @pl.when(c == 0)
    def _():
        pltpu.sync_copy(acc.at[pl.ds(r0, _ZR)], out_hbm.at[0, pl.ds(r0, _ZR)])

    @pl.when(c == 1)
    def _():
        pltpu.sync_copy(acc.at[pl.ds(r0, _ZR)], out_hbm.at[1, pl.ds(r0, _ZR)])


def _make_deg():
    return pl.kernel(
        _deg_body,
        out_type=jax.ShapeDtypeStruct((_NC, _NACC, 128), jnp.float32),
        mesh=_mesh,
        scratch_types=[
            pltpu.VMEM((_BK,), jnp.int32),
            pltpu.VMEM((_BK, 128), jnp.float32),
            pltpu.VMEM((_BK, 128), jnp.float32),
            pltpu.VMEM_SHARED((_NACC, 128), jnp.float32),
            pltpu.SemaphoreType.DMA,
        ],
    )


def _agg_body(nchunks, fc, *refs):
    g_refs = refs[:nchunks]
    src_hbm, dst_hbm, out_hbm = refs[nchunks:nchunks + 3]
    src0, dst0, src1, dst1, rows0, rows1, acc, sem0, sem1 = refs[nchunks + 3:]
    c = lax.axis_index("c")
    s = lax.axis_index("s")
    is_fast = c == _FAST
    base = jnp.where(is_fast, s * _EPW_F, _NS * _EPW_F + s * _EPW_S)
    nb = jnp.where(is_fast, _EPW_F // _BK, _EPW_S // _BK)
    r0 = s * _ZR

    for ci in range(nchunks):
        g_hbm = g_refs[ci]

        def zfill(i, carry):
            for j in range(fc // 16):
                rows0[i, pl.ds(j * 16, 16)] = jnp.zeros((16,), jnp.float32)
            return carry
        lax.fori_loop(0, _BK, zfill, None)
        for k in range(_ZR // _BK):
            pltpu.sync_copy(rows0, acc.at[pl.ds(r0 + k * _BK, _BK)])
        plsc.subcore_barrier()

        pltpu.sync_copy(src_hbm.at[pl.ds(base, _BK)], src0)
        pltpu.sync_copy(dst_hbm.at[pl.ds(base, _BK)], dst0)
        pltpu.async_copy(g_hbm.at[src0], rows0, sem0)

        @pl.loop(0, nb, step=2)
        def _(i):
            pltpu.sync_copy(src_hbm.at[pl.ds(base + (i + 1) * _BK, _BK)],
                            src1)
            pltpu.sync_copy(dst_hbm.at[pl.ds(base + (i + 1) * _BK, _BK)],
                            dst1)
            pltpu.async_copy(g_hbm.at[src1], rows1, sem1)
            pltpu.make_async_copy(g_hbm.at[src0], rows0, sem0).wait()
            pltpu.sync_copy(rows0, acc.at[dst0], add=True)

            @pl.when(i + 2 < nb)
            def _():
                pltpu.sync_copy(src_hbm.at[pl.ds(base + (i + 2) * _BK, _BK)],
                                src0)
                pltpu.sync_copy(dst_hbm.at[pl.ds(base + (i + 2) * _BK, _BK)],
                                dst0)
                pltpu.async_copy(g_hbm.at[src0], rows0, sem0)

            pltpu.make_async_copy(g_hbm.at[src1], rows1, sem1).wait()
            pltpu.sync_copy(rows1, acc.at[dst1], add=True)

        plsc.subcore_barrier()

        @pl.when(c == 0)
        def _(ci=ci):
            pltpu.sync_copy(acc.at[pl.ds(r0, _ZR)],
                            out_hbm.at[ci, 0, pl.ds(r0, _ZR)])

        @pl.when(c == 1)
        def _(ci=ci):
            pltpu.sync_copy(acc.at[pl.ds(r0, _ZR)],
                            out_hbm.at[ci, 1, pl.ds(r0, _ZR)])


def _make_agg(nchunks, fc):
    return pl.kernel(
        functools.partial(_agg_body, nchunks, fc),
        out_type=jax.ShapeDtypeStruct((nchunks, _NC, _NACC, fc), jnp.float32),
        mesh=_mesh,
        scratch_types=[
            pltpu.VMEM((_BK,), jnp.int32),
            pltpu.VMEM((_BK,), jnp.int32),
            pltpu.VMEM((_BK,), jnp.int32),
            pltpu.VMEM((_BK,), jnp.int32),
            pltpu.VMEM((_BK, fc), jnp.float32),
            pltpu.VMEM((_BK, fc), jnp.float32),
            pltpu.VMEM_SHARED((_NACC, fc), jnp.float32),
            pltpu.SemaphoreType.DMA,
            pltpu.SemaphoreType.DMA,
        ],
    )


def _disg0_kernel(degp, x, dis, g0):
    d = degp[0, :, 0:1] + degp[1, :, 0:1] + 1.0
    r = lax.rsqrt(d)
    dis[...] = r
    g0[...] = r * x[...]


def _mm1_kernel(p, g0, dis, w, b, o):
    a = dis[...] * (p[0, 0] + p[0, 1] + g0[...])
    h = jnp.dot(a.astype(jnp.bfloat16), w[...],
                preferred_element_type=jnp.float32) + b[...]
    o[...] = jnp.maximum(h, 0.0).astype(jnp.bfloat16)


def _mm2_kernel(h1, w, dis, o):
    o[0] = dis[...] * jnp.dot(h1[...], w[...],
                              preferred_element_type=jnp.float32)


def _fin2_kernel(p, g2, dis, b, o):
    t = p[0, 0] + p[0, 1] + g2[0]
    o[...] = jnp.maximum(dis[...] * t + b[0], 0.0).astype(jnp.bfloat16)


def _mm3_kernel(h2, w, dis, o):
    o[...] = dis[...] * jnp.dot(h2[...], w[...],
                                preferred_element_type=jnp.float32)


def _mm4_kernel(p, g3, dis, b, w, o):
    h3 = jnp.maximum(dis[...] * (p[0, 0] + p[0, 1] + g3[...]) + b[...], 0.0)
    o[...] = dis[...] * jnp.dot(h3.astype(jnp.bfloat16), w[...],
                                preferred_element_type=jnp.float32)


def _final_kernel(p, g4, dis, b, o):
    t = dis[...] * (p[0, 0] + p[0, 1] + g4[...])
    o[...] = t[:, :64] + b[...]


def kernel(x, edge_index, W1, b1, W2, b2, W3, b3, W4, b4):
    f32 = jnp.float32
    src = edge_index[0]
    dst = edge_index[1]

    pad = _EPAD - _E
    src_p = jnp.concatenate([src, jnp.zeros((pad,), jnp.int32)])
    dst_p = jnp.concatenate([dst, jnp.full((pad,), _N, jnp.int32)])

    bf16 = jnp.bfloat16
    W1b = W1.astype(bf16)
    W2p = jnp.pad(W2, ((0, 0), (0, 12))).astype(bf16)
    b2p = jnp.pad(b2, (0, 12)).reshape(4, 1, 128)
    W3p = jnp.pad(W3, ((0, 12), (0, 28))).astype(bf16)
    b3p = jnp.pad(b3, (0, 28)).reshape(1, 128)
    W4p = jnp.pad(W4, ((0, 28), (0, 64))).astype(bf16)
    b1r = b1.reshape(1, 2000)
    b4r = b4.reshape(1, 64)

    degp = _make_deg()(dst_p)
    dis, g0 = pl.pallas_call(
        _disg0_kernel,
        grid=(25,),
        in_specs=[
            pl.BlockSpec((2, 400, 128), lambda m: (0, m, 0)),
            pl.BlockSpec((400, 128), lambda m: (m, 0)),
        ],
        out_specs=[
            pl.BlockSpec((400, 1), lambda m: (m, 0)),
            pl.BlockSpec((400, 128), lambda m: (m, 0)),
        ],
        out_shape=[
            jax.ShapeDtypeStruct((_N, 1), f32),
            jax.ShapeDtypeStruct((_N, 128), f32),
        ],
    )(degp, x)

    p1 = _make_agg(1, 128)(g0, src_p, dst_p)
    h1 = pl.pallas_call(
        _mm1_kernel,
        grid=(25,),
        in_specs=[
            pl.BlockSpec((1, 2, 400, 128), lambda m: (0, 0, m, 0)),
            pl.BlockSpec((400, 128), lambda m: (m, 0)),
            pl.BlockSpec((400, 1), lambda m: (m, 0)),
            pl.BlockSpec((128, 2000), lambda m: (0, 0)),
            pl.BlockSpec((1, 2000), lambda m: (0, 0)),
        ],
        out_specs=pl.BlockSpec((400, 2000), lambda m: (m, 0)),
        out_shape=jax.ShapeDtypeStruct((_N, 2000), bf16),
    )(p1, g0, dis, W1b, b1r)

    g2 = pl.pallas_call(
        _mm2_kernel,
        grid=(25, 4),
        in_specs=[
            pl.BlockSpec((400, 2000), lambda m, c: (m, 0)),
            pl.BlockSpec((2000, 128), lambda m, c: (0, c)),
            pl.BlockSpec((400, 1), lambda m, c: (m, 0)),
        ],
        out_specs=pl.BlockSpec((1, 400, 128), lambda m, c: (c, m, 0)),
        out_shape=jax.ShapeDtypeStruct((4, _N, 128), f32),
    )(h1, W2p, dis)
    p2 = _make_agg(4, 128)(g2[0], g2[1], g2[2], g2[3], src_p, dst_p)
    h2 = pl.pallas_call(
        _fin2_kernel,
        grid=(4, 25),
        in_specs=[
            pl.BlockSpec((1, 2, 400, 128), lambda c, m: (c, 0, m, 0)),
            pl.BlockSpec((1, 400, 128), lambda c, m: (c, m, 0)),
            pl.BlockSpec((400, 1), lambda c, m: (m, 0)),
            pl.BlockSpec((1, 1, 128), lambda c, m: (c, 0, 0)),
        ],
        out_specs=pl.BlockSpec((400, 128), lambda c, m: (m, c)),
        out_shape=jax.ShapeDtypeStruct((_N, 512), bf16),
    )(p2, g2, dis, b2p)

    g3 = pl.pallas_call(
        _mm3_kernel,
        grid=(25,),
        in_specs=[
            pl.BlockSpec((400, 512), lambda m: (m, 0)),
            pl.BlockSpec((512, 128), lambda m: (0, 0)),
            pl.BlockSpec((400, 1), lambda m: (m, 0)),
        ],
        out_specs=pl.BlockSpec((400, 128), lambda m: (m, 0)),
        out_shape=jax.ShapeDtypeStruct((_N, 128), f32),
    )(h2, W3p, dis)
    p3 = _make_agg(1, 128)(g3, src_p, dst_p)

    g4 = pl.pallas_call(
        _mm4_kernel,
        grid=(25,),
        in_specs=[
            pl.BlockSpec((1, 2, 400, 128), lambda m: (0, 0, m, 0)),
            pl.BlockSpec((400, 128), lambda m: (m, 0)),
            pl.BlockSpec((400, 1), lambda m: (m, 0)),
            pl.BlockSpec((1, 128), lambda m: (0, 0)),
            pl.BlockSpec((128, 128), lambda m: (0, 0)),
        ],
        out_specs=pl.BlockSpec((400, 128), lambda m: (m, 0)),
        out_shape=jax.ShapeDtypeStruct((_N, 128), f32),
    )(p3, g3, dis, b3p, W4p)
    p4 = _make_agg(1, 128)(g4, src_p, dst_p)

    out = pl.pallas_call(
        _final_kernel,
        grid=(25,),
        in_specs=[
            pl.BlockSpec((1, 2, 400, 128), lambda m: (0, 0, m, 0)),
            pl.BlockSpec((400, 128), lambda m: (m, 0)),
            pl.BlockSpec((400, 1), lambda m: (m, 0)),
            pl.BlockSpec((1, 64), lambda m: (0, 0)),
        ],
        out_specs=pl.BlockSpec((400, 64), lambda m: (m, 0)),
        out_shape=jax.ShapeDtypeStruct((_N, 64), f32),
    )(p4, g4, dis, b4r)
    return out


# speedup vs baseline: 1.0641x; 1.0334x over previous
import functools

import jax
import jax.numpy as jnp
from jax import lax
from jax.experimental import pallas as pl
from jax.experimental.pallas import tpu as pltpu
from jax.experimental.pallas import tpu_sc as plsc

_N = 10000
_E = 160000
_NC, _NS = 2, 16
_NW = _NC * _NS
_BK = 128
_EPAD = 163840
_FAST = 1
_EPW_F = 9472
_EPW_S = 768
_EPW_D = 5120
_NACC = 10240
_ZR = _NACC // _NS

_mesh = plsc.VectorSubcoreMesh(core_axis_name="c", subcore_axis_name="s")


def _deg_body(dst_hbm, out_hbm, dst_v, ones_v, zb, acc, sem):
    del sem
    c = lax.axis_index("c")
    s = lax.axis_index("s")
    base = (s * _NC + c) * _EPW_D
    nb = _EPW_D // _BK
    r0 = s * _ZR

    def fill(i, carry):
        for j in range(8):
            zb[i, pl.ds(j * 16, 16)] = jnp.zeros((16,), jnp.float32)
            ones_v[i, pl.ds(j * 16, 16)] = jnp.ones((16,), jnp.float32)
        return carry
    lax.fori_loop(0, _BK, fill, None)

    for k in range(_ZR // _BK):
        pltpu.sync_copy(zb, acc.at[pl.ds(r0 + k * _BK, _BK)])
    plsc.subcore_barrier()

    def estep(i, carry):
        pltpu.sync_copy(dst_hbm.at[pl.ds(base + i * _BK, _BK)], dst_v)
        pltpu.sync_copy(ones_v, acc.at[dst_v], add=True)
        return carry
    lax.fori_loop(0, nb, estep, None)

    plsc.subcore_barrier()

    @pl.when(c == 0)
    def _():
        pltpu.sync_copy(acc.at[pl.ds(r0, _ZR)], out_hbm.at[0, pl.ds(r0, _ZR)])

    @pl.when(c == 1)
    def _():
        pltpu.sync_copy(acc.at[pl.ds(r0, _ZR)], out_hbm.at[1, pl.ds(r0, _ZR)])


def _make_deg():
    return pl.kernel(
        _deg_body,
        out_type=jax.ShapeDtypeStruct((_NC, _NACC, 128), jnp.float32),
        mesh=_mesh,
        scratch_types=[
            pltpu.VMEM((_BK,), jnp.int32),
            pltpu.VMEM((_BK, 128), jnp.float32),
            pltpu.VMEM((_BK, 128), jnp.float32),
            pltpu.VMEM_SHARED((_NACC, 128), jnp.float32),
            pltpu.SemaphoreType.DMA,
        ],
    )


def _agg_body(nchunks, fc, *refs):
    g_refs = refs[:nchunks]
    src_hbm, dst_hbm, out_hbm = refs[nchunks:nchunks + 3]
    src0, dst0, src1, dst1, rows0, rows1, acc, sem0, sem1 = refs[nchunks + 3:]
    c = lax.axis_index("c")
    s = lax.axis_index("s")
    is_fast = c == _FAST
    base = jnp.where(is_fast, s * _EPW_F, _NS * _EPW_F + s * _EPW_S)
    nb = jnp.where(is_fast, _EPW_F // _BK, _EPW_S // _BK)
    r0 = s * _ZR

    for ci in range(nchunks):
        g_hbm = g_refs[ci]

        def zfill(i, carry):
            for j in range(fc // 16):
                rows0[i, pl.ds(j * 16, 16)] = jnp.zeros((16,), jnp.float32)
            return carry
        lax.fori_loop(0, _BK, zfill, None)
        for k in range(_ZR // _BK):
            pltpu.sync_copy(rows0, acc.at[pl.ds(r0 + k * _BK, _BK)])
        plsc.subcore_barrier()

        pltpu.sync_copy(src_hbm.at[pl.ds(base, _BK)], src0)
        pltpu.sync_copy(dst_hbm.at[pl.ds(base, _BK)], dst0)
        pltpu.async_copy(g_hbm.at[src0], rows0, sem0)

        @pl.loop(0, nb, step=2)
        def _(i):
            pltpu.sync_copy(src_hbm.at[pl.ds(base + (i + 1) * _BK, _BK)],
                            src1)
            pltpu.sync_copy(dst_hbm.at[pl.ds(base + (i + 1) * _BK, _BK)],
                            dst1)
            pltpu.async_copy(g_hbm.at[src1], rows1, sem1)
            pltpu.make_async_copy(g_hbm.at[src0], rows0, sem0).wait()
            pltpu.sync_copy(rows0, acc.at[dst0], add=True)

            @pl.when(i + 2 < nb)
            def _():
                pltpu.sync_copy(src_hbm.at[pl.ds(base + (i + 2) * _BK, _BK)],
                                src0)
                pltpu.sync_copy(dst_hbm.at[pl.ds(base + (i + 2) * _BK, _BK)],
                                dst0)
                pltpu.async_copy(g_hbm.at[src0], rows0, sem0)

            pltpu.make_async_copy(g_hbm.at[src1], rows1, sem1).wait()
            pltpu.sync_copy(rows1, acc.at[dst1], add=True)

        plsc.subcore_barrier()

        @pl.when(c == 0)
        def _(ci=ci):
            pltpu.sync_copy(acc.at[pl.ds(r0, _ZR)],
                            out_hbm.at[ci, 0, pl.ds(r0, _ZR)])

        @pl.when(c == 1)
        def _(ci=ci):
            pltpu.sync_copy(acc.at[pl.ds(r0, _ZR)],
                            out_hbm.at[ci, 1, pl.ds(r0, _ZR)])


def _make_agg(nchunks, fc):
    return pl.kernel(
        functools.partial(_agg_body, nchunks, fc),
        out_type=jax.ShapeDtypeStruct((nchunks, _NC, _NACC, fc), jnp.float32),
        mesh=_mesh,
        scratch_types=[
            pltpu.VMEM((_BK,), jnp.int32),
            pltpu.VMEM((_BK,), jnp.int32),
            pltpu.VMEM((_BK,), jnp.int32),
            pltpu.VMEM((_BK,), jnp.int32),
            pltpu.VMEM((_BK, fc), jnp.float32),
            pltpu.VMEM((_BK, fc), jnp.float32),
            pltpu.VMEM_SHARED((_NACC, fc), jnp.float32),
            pltpu.SemaphoreType.DMA,
            pltpu.SemaphoreType.DMA,
        ],
    )


def _disg0_kernel(degp, x, dis, g0):
    d = degp[0, :, 0:1] + degp[1, :, 0:1] + 1.0
    r = lax.rsqrt(d)
    dis[...] = r
    g0[...] = r * x[...]


def _mm1_kernel(p, g0, dis, w, b, o):
    a = dis[...] * (p[0, 0] + p[0, 1] + g0[...])
    h = jnp.dot(a.astype(jnp.bfloat16), w[...],
                preferred_element_type=jnp.float32) + b[...]
    o[...] = jnp.maximum(h, 0.0).astype(jnp.bfloat16)


def _mm2_kernel(h1, w, dis, o):
    o[0] = dis[...] * jnp.dot(h1[...], w[...],
                              preferred_element_type=jnp.float32)


def _fin2_kernel(p, g2, dis, b, o):
    t = p[0, 0] + p[0, 1] + g2[0]
    o[...] = jnp.maximum(dis[...] * t + b[0], 0.0).astype(jnp.bfloat16)


def _mm3_kernel(h2, w, dis, o):
    o[...] = dis[...] * jnp.dot(h2[...], w[...],
                                preferred_element_type=jnp.float32)


def _mm4_kernel(p, g3, dis, b, w, o):
    h3 = jnp.maximum(dis[...] * (p[0, 0] + p[0, 1] + g3[...]) + b[...], 0.0)
    o[...] = dis[...] * jnp.dot(h3.astype(jnp.bfloat16), w[...],
                                preferred_element_type=jnp.float32)


def _final_kernel(p, g4, dis, b, o):
    t = dis[...] * (p[0, 0] + p[0, 1] + g4[...])
    o[...] = t[:, :64] + b[...]


def kernel(x, edge_index, W1, b1, W2, b2, W3, b3, W4, b4):
    f32 = jnp.float32
    src = edge_index[0]
    dst = edge_index[1]

    pad = _EPAD - _E
    src_p = jnp.concatenate([src, jnp.zeros((pad,), jnp.int32)])
    dst_p = jnp.concatenate([dst, jnp.full((pad,), _N, jnp.int32)])

    bf16 = jnp.bfloat16
    W1b = W1.astype(bf16)
    W2p = jnp.pad(W2, ((0, 0), (0, 12))).astype(bf16)
    b2p = jnp.pad(b2, (0, 12)).reshape(4, 1, 128)
    W3p = jnp.pad(W3, ((0, 12), (0, 28))).astype(bf16)
    b3p = jnp.pad(b3, (0, 28)).reshape(1, 128)
    W4p = jnp.pad(W4, ((0, 28), (0, 64))).astype(bf16)
    b1r = b1.reshape(1, 2000)
    b4r = b4.reshape(1, 64)

    degp = _make_deg()(dst_p)
    dis, g0 = pl.pallas_call(
        _disg0_kernel,
        grid=(25,),
        in_specs=[
            pl.BlockSpec((2, 400, 128), lambda m: (0, m, 0)),
            pl.BlockSpec((400, 128), lambda m: (m, 0)),
        ],
        out_specs=[
            pl.BlockSpec((400, 1), lambda m: (m, 0)),
            pl.BlockSpec((400, 128), lambda m: (m, 0)),
        ],
        out_shape=[
            jax.ShapeDtypeStruct((_N, 1), f32),
            jax.ShapeDtypeStruct((_N, 128), f32),
        ],
    )(degp, x)

    p1 = _make_agg(1, 128)(g0, src_p, dst_p)
    h1 = pl.pallas_call(
        _mm1_kernel,
        grid=(25,),
        in_specs=[
            pl.BlockSpec((1, 2, 400, 128), lambda m: (0, 0, m, 0)),
            pl.BlockSpec((400, 128), lambda m: (m, 0)),
            pl.BlockSpec((400, 1), lambda m: (m, 0)),
            pl.BlockSpec((128, 2000), lambda m: (0, 0)),
            pl.BlockSpec((1, 2000), lambda m: (0, 0)),
        ],
        out_specs=pl.BlockSpec((400, 2000), lambda m: (m, 0)),
        out_shape=jax.ShapeDtypeStruct((_N, 2000), bf16),
    )(p1, g0, dis, W1b, b1r)

    g2 = pl.pallas_call(
        _mm2_kernel,
        grid=(25, 4),
        in_specs=[
            pl.BlockSpec((400, 2000), lambda m, c: (m, 0)),
            pl.BlockSpec((2000, 128), lambda m, c: (0, c)),
            pl.BlockSpec((400, 1), lambda m, c: (m, 0)),
        ],
        out_specs=pl.BlockSpec((1, 400, 128), lambda m, c: (c, m, 0)),
        out_shape=jax.ShapeDtypeStruct((4, _N, 128), f32),
    )(h1, W2p, dis)
    p2 = _make_agg(4, 128)(g2[0], g2[1], g2[2], g2[3], src_p, dst_p)
    h2 = pl.pallas_call(
        _fin2_kernel,
        grid=(4, 25),
        in_specs=[
            pl.BlockSpec((1, 2, 400, 128), lambda c, m: (c, 0, m, 0)),
            pl.BlockSpec((1, 400, 128), lambda c, m: (c, m, 0)),
            pl.BlockSpec((400, 1), lambda c, m: (m, 0)),
            pl.BlockSpec((1, 1, 128), lambda c, m: (c, 0, 0)),
        ],
        out_specs=pl.BlockSpec((400, 128), lambda c, m: (m, c)),
        out_shape=jax.ShapeDtypeStruct((_N, 512), bf16),
    )(p2, g2, dis, b2p)

    g3 = pl.pallas_call(
        _mm3_kernel,
        grid=(25,),
        in_specs=[
            pl.BlockSpec((400, 512), lambda m: (m, 0)),
            pl.BlockSpec((512, 128), lambda m: (0, 0)),
            pl.BlockSpec((400, 1), lambda m: (m, 0)),
        ],
        out_specs=pl.BlockSpec((400, 128), lambda m: (m, 0)),
        out_shape=jax.ShapeDtypeStruct((_N, 128), f32),
    )(h2, W3p, dis)
    p3 = _make_agg(1, 128)(g3, src_p, dst_p)

    g4 = pl.pallas_call(
        _mm4_kernel,
        grid=(25,),
        in_specs=[
            pl.BlockSpec((1, 2, 400, 128), lambda m: (0, 0, m, 0)),
            pl.BlockSpec((400, 128), lambda m: (m, 0)),
            pl.BlockSpec((400, 1), lambda m: (m, 0)),
            pl.BlockSpec((1, 128), lambda m: (0, 0)),
            pl.BlockSpec((128, 128), lambda m: (0, 0)),
        ],
        out_specs=pl.BlockSpec((400, 128), lambda m: (m, 0)),
        out_shape=jax.ShapeDtypeStruct((_N, 128), f32),
    )(p3, g3, dis, b3p, W4p)
    p4 = _make_agg(1, 128)(g4, src_p, dst_p)

    out = pl.pallas_call(
        _final_kernel,
        grid=(25,),
        in_specs=[
            pl.BlockSpec((1, 2, 400, 128), lambda m: (0, 0, m, 0)),
            pl.BlockSpec((400, 128), lambda m: (m, 0)),
            pl.BlockSpec((400, 1), lambda m: (m, 0)),
            pl.BlockSpec((1, 64), lambda m: (0, 0)),
        ],
        out_specs=pl.BlockSpec((400, 64), lambda m: (m, 0)),
        out_shape=jax.ShapeDtypeStruct((_N, 64), f32),
    )(p4, g4, dis, b4r)
    return out


# speedup vs baseline: 1.1080x; 1.0413x over previous
import functools

import jax
import jax.numpy as jnp
from jax import lax
from jax.experimental import pallas as pl
from jax.experimental.pallas import tpu as pltpu
from jax.experimental.pallas import tpu_sc as plsc

_N = 10000
_E = 160000
_NC, _NS = 2, 16
_NW = _NC * _NS
_BK = 128
_EPAD = 163840
_FAST = 1
_EPW_F = 9472
_EPW_S = 768
_EPW_D = 5120
_NACC = 10240
_ZR = _NACC // _NS

_mesh = plsc.VectorSubcoreMesh(core_axis_name="c", subcore_axis_name="s")


def _deg_body(dst_hbm, out_hbm, dst_v, ones_v, zb, acc, sem):
    del sem
    c = lax.axis_index("c")
    s = lax.axis_index("s")
    base = (s * _NC + c) * _EPW_D
    nb = _EPW_D // _BK
    r0 = s * _ZR

    def fill(i, carry):
        for j in range(8):
            zb[i, pl.ds(j * 16, 16)] = jnp.zeros((16,), jnp.float32)
            ones_v[i, pl.ds(j * 16, 16)] = jnp.ones((16,), jnp.float32)
        return carry
    lax.fori_loop(0, _BK, fill, None)

    for k in range(_ZR // _BK):
        pltpu.sync_copy(zb, acc.at[pl.ds(r0 + k * _BK, _BK)])
    plsc.subcore_barrier()

    def estep(i, carry):
        pltpu.sync_copy(dst_hbm.at[pl.ds(base + i * _BK, _BK)], dst_v)
        pltpu.sync_copy(ones_v, acc.at[dst_v], add=True)
        return carry
    lax.fori_loop(0, nb, estep, None)

    plsc.subcore_barrier()

    @pl.when(c == 0)
    def _():
        pltpu.sync_copy(acc.at[pl.ds(r0, _ZR)], out_hbm.at[0, pl.ds(r0, _ZR)])

    @pl.when(c == 1)
    def _():
        pltpu.sync_copy(acc.at[pl.ds(r0, _ZR)], out_hbm.at[1, pl.ds(r0, _ZR)])


def _make_deg():
    return pl.kernel(
        _deg_body,
        out_type=jax.ShapeDtypeStruct((_NC, _NACC, 128), jnp.float32),
        mesh=_mesh,
        scratch_types=[
            pltpu.VMEM((_BK,), jnp.int32),
            pltpu.VMEM((_BK, 128), jnp.float32),
            pltpu.VMEM((_BK, 128), jnp.float32),
            pltpu.VMEM_SHARED((_NACC, 128), jnp.float32),
            pltpu.SemaphoreType.DMA,
        ],
    )


def _agg_body(nchunks, fc, *refs):
    g_refs = refs[:nchunks]
    src_hbm, dst_hbm, out_hbm = refs[nchunks:nchunks + 3]
    src0, dst0, src1, dst1, rows0, rows1, acc, sem0, sem1 = refs[nchunks + 3:]
    c = lax.axis_index("c")
    s = lax.axis_index("s")
    is_fast = c == _FAST
    base = jnp.where(is_fast, s * _EPW_F, _NS * _EPW_F + s * _EPW_S)
    nb = jnp.where(is_fast, _EPW_F // _BK, _EPW_S // _BK)
    r0 = s * _ZR

    for ci in range(nchunks):
        g_hbm = g_refs[ci]

        def zfill(i, carry):
            for j in range(fc // 16):
                rows0[i, pl.ds(j * 16, 16)] = jnp.zeros((16,), jnp.float32)
            return carry
        lax.fori_loop(0, _BK, zfill, None)
        for k in range(_ZR // _BK):
            pltpu.sync_copy(rows0, acc.at[pl.ds(r0 + k * _BK, _BK)])
        plsc.subcore_barrier()

        pltpu.sync_copy(src_hbm.at[pl.ds(base, _BK)], src0)
        pltpu.sync_copy(dst_hbm.at[pl.ds(base, _BK)], dst0)
        pltpu.async_copy(g_hbm.at[src0], rows0, sem0)

        @pl.loop(0, nb, step=2)
        def _(i):
            pltpu.sync_copy(src_hbm.at[pl.ds(base + (i + 1) * _BK, _BK)],
                            src1)
            pltpu.sync_copy(dst_hbm.at[pl.ds(base + (i + 1) * _BK, _BK)],
                            dst1)
            pltpu.async_copy(g_hbm.at[src1], rows1, sem1)
            pltpu.make_async_copy(g_hbm.at[src0], rows0, sem0).wait()
            pltpu.sync_copy(rows0, acc.at[dst0], add=True)

            @pl.when(i + 2 < nb)
            def _():
                pltpu.sync_copy(src_hbm.at[pl.ds(base + (i + 2) * _BK, _BK)],
                                src0)
                pltpu.sync_copy(dst_hbm.at[pl.ds(base + (i + 2) * _BK, _BK)],
                                dst0)
                pltpu.async_copy(g_hbm.at[src0], rows0, sem0)

            pltpu.make_async_copy(g_hbm.at[src1], rows1, sem1).wait()
            pltpu.sync_copy(rows1, acc.at[dst1], add=True)

        plsc.subcore_barrier()

        @pl.when(c == 0)
        def _(ci=ci):
            pltpu.sync_copy(acc.at[pl.ds(r0, _ZR)],
                            out_hbm.at[ci, 0, pl.ds(r0, _ZR)])

        @pl.when(c == 1)
        def _(ci=ci):
            pltpu.sync_copy(acc.at[pl.ds(r0, _ZR)],
                            out_hbm.at[ci, 1, pl.ds(r0, _ZR)])


def _make_agg(nchunks, fc):
    return pl.kernel(
        functools.partial(_agg_body, nchunks, fc),
        out_type=jax.ShapeDtypeStruct((nchunks, _NC, _NACC, fc), jnp.float32),
        mesh=_mesh,
        scratch_types=[
            pltpu.VMEM((_BK,), jnp.int32),
            pltpu.VMEM((_BK,), jnp.int32),
            pltpu.VMEM((_BK,), jnp.int32),
            pltpu.VMEM((_BK,), jnp.int32),
            pltpu.VMEM((_BK, fc), jnp.float32),
            pltpu.VMEM((_BK, fc), jnp.float32),
            pltpu.VMEM_SHARED((_NACC, fc), jnp.float32),
            pltpu.SemaphoreType.DMA,
            pltpu.SemaphoreType.DMA,
        ],
    )


def _disg0_kernel(degp, x, dis, g0):
    d = degp[0, :, 0:1] + degp[1, :, 0:1] + 1.0
    r = lax.rsqrt(d)
    dis[...] = r
    g0[...] = r * x[...]


def _mm1_kernel(p, g0, dis, w, b, o):
    a = dis[...] * (p[0, 0] + p[0, 1] + g0[...])
    h = jnp.dot(a.astype(jnp.bfloat16), w[...],
                preferred_element_type=jnp.float32) + b[...]
    o[...] = jnp.maximum(h, 0.0).astype(jnp.bfloat16)


def _mm2_kernel(h1, w, dis, o):
    o[0] = dis[...] * jnp.dot(h1[...], w[...],
                              preferred_element_type=jnp.float32)


def _fin2_kernel(p, g2, dis, b, o):
    t = p[0, 0] + p[0, 1] + g2[0]
    o[...] = jnp.maximum(dis[...] * t + b[...], 0.0).astype(jnp.bfloat16)


def _mm3_kernel(h2a, h2b, h2c, h2d, w, dis, o):
    h2 = jnp.concatenate([h2a[...], h2b[...], h2c[...], h2d[...]], axis=1)
    o[...] = dis[...] * jnp.dot(h2, w[...],
                                preferred_element_type=jnp.float32)


def _mm4_kernel(p, g3, dis, b, w, o):
    h3 = jnp.maximum(dis[...] * (p[0, 0] + p[0, 1] + g3[...]) + b[...], 0.0)
    o[...] = dis[...] * jnp.dot(h3.astype(jnp.bfloat16), w[...],
                                preferred_element_type=jnp.float32)


def _final_kernel(p, g4, dis, b, o):
    t = dis[...] * (p[0, 0] + p[0, 1] + g4[...])
    o[...] = t[:, :64] + b[...]


def kernel(x, edge_index, W1, b1, W2, b2, W3, b3, W4, b4):
    f32 = jnp.float32
    src = edge_index[0]
    dst = edge_index[1]

    pad = _EPAD - _E
    src_p = jnp.concatenate([src, jnp.zeros((pad,), jnp.int32)])
    dst_p = jnp.concatenate([dst, jnp.full((pad,), _N, jnp.int32)])

    bf16 = jnp.bfloat16
    W1b = W1.astype(bf16)
    W2p = jnp.pad(W2, ((0, 0), (0, 12))).astype(bf16)
    b2p = jnp.pad(b2, (0, 12)).reshape(4, 128)
    W3p = jnp.pad(W3, ((0, 12), (0, 28))).astype(bf16)
    b3p = jnp.pad(b3, (0, 28)).reshape(1, 128)
    W4p = jnp.pad(W4, ((0, 28), (0, 64))).astype(bf16)
    b1r = b1.reshape(1, 2000)
    b4r = b4.reshape(1, 64)

    degp = _make_deg()(dst_p)
    dis, g0 = pl.pallas_call(
        _disg0_kernel,
        grid=(25,),
        in_specs=[
            pl.BlockSpec((2, 400, 128), lambda m: (0, m, 0)),
            pl.BlockSpec((400, 128), lambda m: (m, 0)),
        ],
        out_specs=[
            pl.BlockSpec((400, 1), lambda m: (m, 0)),
            pl.BlockSpec((400, 128), lambda m: (m, 0)),
        ],
        out_shape=[
            jax.ShapeDtypeStruct((_N, 1), f32),
            jax.ShapeDtypeStruct((_N, 128), f32),
        ],
    )(degp, x)

    p1 = _make_agg(1, 128)(g0, src_p, dst_p)
    h1 = pl.pallas_call(
        _mm1_kernel,
        grid=(25,),
        in_specs=[
            pl.BlockSpec((1, 2, 400, 128), lambda m: (0, 0, m, 0)),
            pl.BlockSpec((400, 128), lambda m: (m, 0)),
            pl.BlockSpec((400, 1), lambda m: (m, 0)),
            pl.BlockSpec((128, 2000), lambda m: (0, 0)),
            pl.BlockSpec((1, 2000), lambda m: (0, 0)),
        ],
        out_specs=pl.BlockSpec((400, 2000), lambda m: (m, 0)),
        out_shape=jax.ShapeDtypeStruct((_N, 2000), bf16),
    )(p1, g0, dis, W1b, b1r)

    mm2 = pl.pallas_call(
        _mm2_kernel,
        grid=(25,),
        in_specs=[
            pl.BlockSpec((400, 2000), lambda m: (m, 0)),
            pl.BlockSpec((2000, 128), lambda m: (0, 0)),
            pl.BlockSpec((400, 1), lambda m: (m, 0)),
        ],
        out_specs=pl.BlockSpec((1, 400, 128), lambda m: (0, m, 0)),
        out_shape=jax.ShapeDtypeStruct((1, _N, 128), f32),
    )
    agg1 = _make_agg(1, 128)
    g2cs = [mm2(h1, W2p[:, ci * 128:(ci + 1) * 128], dis) for ci in range(4)]
    p2cs = [agg1(g2c[0], src_p, dst_p) for g2c in g2cs]
    fin2 = pl.pallas_call(
        _fin2_kernel,
        grid=(25,),
        in_specs=[
            pl.BlockSpec((1, 2, 400, 128), lambda m: (0, 0, m, 0)),
            pl.BlockSpec((1, 400, 128), lambda m: (0, m, 0)),
            pl.BlockSpec((400, 1), lambda m: (m, 0)),
            pl.BlockSpec((1, 128), lambda m: (0, 0)),
        ],
        out_specs=pl.BlockSpec((400, 128), lambda m: (m, 0)),
        out_shape=jax.ShapeDtypeStruct((_N, 128), bf16),
    )
    h2cs = [fin2(p2cs[ci], g2cs[ci], dis, b2p[ci:ci + 1])
            for ci in range(4)]

    g3 = pl.pallas_call(
        _mm3_kernel,
        grid=(25,),
        in_specs=[
            pl.BlockSpec((400, 128), lambda m: (m, 0)),
            pl.BlockSpec((400, 128), lambda m: (m, 0)),
            pl.BlockSpec((400, 128), lambda m: (m, 0)),
            pl.BlockSpec((400, 128), lambda m: (m, 0)),
            pl.BlockSpec((512, 128), lambda m: (0, 0)),
            pl.BlockSpec((400, 1), lambda m: (m, 0)),
        ],
        out_specs=pl.BlockSpec((400, 128), lambda m: (m, 0)),
        out_shape=jax.ShapeDtypeStruct((_N, 128), f32),
    )(h2cs[0], h2cs[1], h2cs[2], h2cs[3], W3p, dis)
    p3 = _make_agg(1, 128)(g3, src_p, dst_p)

    g4 = pl.pallas_call(
        _mm4_kernel,
        grid=(25,),
        in_specs=[
            pl.BlockSpec((1, 2, 400, 128), lambda m: (0, 0, m, 0)),
            pl.BlockSpec((400, 128), lambda m: (m, 0)),
            pl.BlockSpec((400, 1), lambda m: (m, 0)),
            pl.BlockSpec((1, 128), lambda m: (0, 0)),
            pl.BlockSpec((128, 128), lambda m: (0, 0)),
        ],
        out_specs=pl.BlockSpec((400, 128), lambda m: (m, 0)),
        out_shape=jax.ShapeDtypeStruct((_N, 128), f32),
    )(p3, g3, dis, b3p, W4p)
    p4 = _make_agg(1, 128)(g4, src_p, dst_p)

    out = pl.pallas_call(
        _final_kernel,
        grid=(25,),
        in_specs=[
            pl.BlockSpec((1, 2, 400, 128), lambda m: (0, 0, m, 0)),
            pl.BlockSpec((400, 128), lambda m: (m, 0)),
            pl.BlockSpec((400, 1), lambda m: (m, 0)),
            pl.BlockSpec((1, 64), lambda m: (0, 0)),
        ],
        out_specs=pl.BlockSpec((400, 64), lambda m: (m, 0)),
        out_shape=jax.ShapeDtypeStruct((_N, 64), f32),
    )(p4, g4, dis, b4r)
    return out
